# baseline (speedup 1.0000x reference)
import functools

import jax
import jax.numpy as jnp
from jax.experimental import pallas as pl
from jax.experimental.pallas import tpu as pltpu

HIDDEN = 4096
NUM_EXPERTS = 64
TOP_K = 8

BT = 2048
BH = 512
NJ = HIDDEN // BH
NQ = 4
QT = BT // NQ

_INTERPRET = False


def _router_kernel(x_hbm, w1_hbm, b1_ref, w2_ref, b2_ref,
                   rw_ref, idx_ref, tkw_ref,
                   x_vmem, w1_buf, acc_ref, sem_x, sem_w, *, n_blocks):
    i = pl.program_id(0)

    def x_copy(q, blk):
        return pltpu.make_async_copy(
            x_hbm.at[pl.ds(blk * BT + q * QT, QT), :],
            x_vmem.at[pl.ds(q * QT, QT), :], sem_x.at[q])

    def w1_copy(jc, buf):
        return pltpu.make_async_copy(
            w1_hbm.at[pl.ds(jc * BH, BH), :], w1_buf.at[buf], sem_w.at[buf])

    @pl.when(i == 0)
    def _():
        w1_copy(0, 0).start()
        for q in range(NQ):
            x_copy(q, 0).start()

    def panel_dot(h_rows, jc):
        h = jax.lax.dot_general(
            h_rows, w1_buf[jc % 2],
            dimension_numbers=(((1,), (1,)), ((), ())),
            preferred_element_type=jnp.float32)
        h = jnp.maximum(h + b1_ref[jc][0, :], 0.0)
        return jax.lax.dot_general(
            w2_ref[jc], h,
            dimension_numbers=(((1,), (1,)), ((), ())),
            preferred_element_type=jnp.float32)

    w1_copy(0, 0).wait()
    w1_copy(1, 1).start()
    for q in range(NQ):
        x_copy(q, i).wait()
        acc_ref[:, pl.ds(q * QT, QT)] = panel_dot(
            x_vmem[pl.ds(q * QT, QT), :], 0)

    for jc in range(1, NJ):
        w1_copy(jc, jc % 2).wait()
        w1_copy((jc + 1) % NJ, (jc + 1) % 2).start()
        acc_ref[...] += panel_dot(x_vmem[...], jc)

    @pl.when(i + 1 < n_blocks)
    def _():
        for q in range(NQ):
            x_copy(q, i + 1).start()

    @pl.when(i == n_blocks - 1)
    def _():
        w1_copy(0, 0).wait()

    logits = acc_ref[...] + b2_ref[:, 0:1]
    m = jnp.max(logits, axis=0, keepdims=True)
    e = jnp.exp(logits - m)
    w = e * (1.0 / jnp.sum(e, axis=0, keepdims=True))
    rw_ref[...] = w

    expert = jax.lax.broadcasted_iota(jnp.int32, (NUM_EXPERTS, BT), 0)
    vals = w
    idx_rows = []
    val_rows = []
    for _ in range(TOP_K):
        mx = jnp.max(vals, axis=0, keepdims=True)
        amx = jnp.min(jnp.where(vals == mx, expert, NUM_EXPERTS),
                      axis=0, keepdims=True)
        idx_rows.append(amx)
        val_rows.append(mx)
        vals = jnp.where(expert == amx, -jnp.inf, vals)
    idx8 = jnp.concatenate(idx_rows, axis=0)
    w8 = jnp.concatenate(val_rows, axis=0)
    idx_ref[...] = idx8
    tkw_ref[...] = w8 * (1.0 / jnp.sum(w8, axis=0, keepdims=True))


def kernel(x, W1, b1, W2, b2):
    B, T, K = x.shape
    n_tok = B * T
    x2 = x.reshape(n_tok, K)
    b1r = b1.reshape(NJ, 1, BH)
    w2r = W2.reshape(NUM_EXPERTS, NJ, BH).transpose(1, 0, 2)
    b2r = b2.reshape(NUM_EXPERTS, 1)

    n_i = n_tok // BT

    out_shapes = (
        jax.ShapeDtypeStruct((NUM_EXPERTS, n_tok), jnp.float32),
        jax.ShapeDtypeStruct((TOP_K, n_tok), jnp.int32),
        jax.ShapeDtypeStruct((TOP_K, n_tok), jnp.float32),
    )

    rw, idx, tkw = pl.pallas_call(
        functools.partial(_router_kernel, n_blocks=n_i),
        grid=(n_i,),
        in_specs=[
            pl.BlockSpec(memory_space=pl.ANY),
            pl.BlockSpec(memory_space=pl.ANY),
            pl.BlockSpec((NJ, 1, BH), lambda i: (0, 0, 0)),
            pl.BlockSpec((NJ, NUM_EXPERTS, BH), lambda i: (0, 0, 0)),
            pl.BlockSpec((NUM_EXPERTS, 1), lambda i: (0, 0)),
        ],
        out_specs=[
            pl.BlockSpec((NUM_EXPERTS, BT), lambda i: (0, i)),
            pl.BlockSpec((TOP_K, BT), lambda i: (0, i)),
            pl.BlockSpec((TOP_K, BT), lambda i: (0, i)),
        ],
        out_shape=out_shapes,
        scratch_shapes=[pltpu.VMEM((BT, HIDDEN), jnp.float32),
                        pltpu.VMEM((2, BH, HIDDEN), jnp.float32),
                        pltpu.VMEM((NUM_EXPERTS, BT), jnp.float32),
                        pltpu.SemaphoreType.DMA((NQ,)),
                        pltpu.SemaphoreType.DMA((2,))],
        compiler_params=pltpu.CompilerParams(
            dimension_semantics=("arbitrary",)),
        interpret=_INTERPRET,
    )(x2, W1, b1r, w2r, b2r)

    return (rw.T.reshape(B, T, NUM_EXPERTS),
            idx.T.reshape(B, T, TOP_K),
            tkw.T.reshape(B, T, TOP_K))


# speedup vs baseline: 1.0056x; 1.0056x over previous
import functools

import jax
import jax.numpy as jnp
from jax.experimental import pallas as pl
from jax.experimental.pallas import tpu as pltpu

HIDDEN = 4096
NUM_EXPERTS = 64
TOP_K = 8

BT = 2048
BH = 512
NJ = HIDDEN // BH
NW = 2
NS = 4
RS = BH // NS
NQ = 4
QT = BT // NQ

_INTERPRET = False


def _router_kernel(x_hbm, w1_hbm, b1_ref, w2_ref, b2_ref,
                   rw_ref, idx_ref, tkw_ref,
                   x_vmem, w1_buf, acc_ref, sem_x, sem_w, *, n_blocks):
    i = pl.program_id(0)

    def x_copy(q, blk):
        return pltpu.make_async_copy(
            x_hbm.at[pl.ds(blk * BT + q * QT, QT), :],
            x_vmem.at[pl.ds(q * QT, QT), :], sem_x.at[q])

    def w1_part(jc, buf, s):
        return pltpu.make_async_copy(
            w1_hbm.at[pl.ds(jc * BH + s * RS, RS), :],
            w1_buf.at[buf, pl.ds(s * RS, RS), :], sem_w.at[buf, s])

    def w1_start(jc, buf):
        for s in range(NS):
            w1_part(jc, buf, s).start()

    def w1_wait(jc, buf):
        for s in range(NS):
            w1_part(jc, buf, s).wait()

    @pl.when(i == 0)
    def _():
        for p in range(NW - 1):
            w1_start(p, p)
        for q in range(NQ):
            x_copy(q, 0).start()

    def panel_dot(h_rows, jc):
        h = jax.lax.dot_general(
            h_rows, w1_buf[jc % NW],
            dimension_numbers=(((1,), (1,)), ((), ())),
            preferred_element_type=jnp.float32)
        h = jnp.maximum(h + b1_ref[jc][0, :], 0.0)
        return jax.lax.dot_general(
            w2_ref[jc], h,
            dimension_numbers=(((1,), (1,)), ((), ())),
            preferred_element_type=jnp.float32)

    w1_wait(0, 0)
    w1_start(NW - 1, NW - 1)
    for q in range(NQ):
        x_copy(q, i).wait()
        acc_ref[:, pl.ds(q * QT, QT)] = panel_dot(
            x_vmem[pl.ds(q * QT, QT), :], 0)

    for jc in range(1, NJ):
        w1_wait(jc, jc % NW)
        w1_start((jc + NW - 1) % NJ, (jc + NW - 1) % NW)
        acc_ref[...] += panel_dot(x_vmem[...], jc)

    @pl.when(i + 1 < n_blocks)
    def _():
        for q in range(NQ):
            x_copy(q, i + 1).start()

    @pl.when(i == n_blocks - 1)
    def _():
        for p in range(NW - 1):
            w1_wait(p, p)

    logits = acc_ref[...] + b2_ref[:, 0:1]
    m = jnp.max(logits, axis=0, keepdims=True)
    e = jnp.exp(logits - m)
    w = e * (1.0 / jnp.sum(e, axis=0, keepdims=True))
    rw_ref[...] = w

    expert = jax.lax.broadcasted_iota(jnp.int32, (NUM_EXPERTS, BT), 0)
    vals = w
    idx_rows = []
    val_rows = []
    for _ in range(TOP_K):
        mx = jnp.max(vals, axis=0, keepdims=True)
        amx = jnp.min(jnp.where(vals == mx, expert, NUM_EXPERTS),
                      axis=0, keepdims=True)
        idx_rows.append(amx)
        val_rows.append(mx)
        vals = jnp.where(expert == amx, -jnp.inf, vals)
    idx8 = jnp.concatenate(idx_rows, axis=0)
    w8 = jnp.concatenate(val_rows, axis=0)
    idx_ref[...] = idx8
    tkw_ref[...] = w8 * (1.0 / jnp.sum(w8, axis=0, keepdims=True))


def kernel(x, W1, b1, W2, b2):
    B, T, K = x.shape
    n_tok = B * T
    x2 = x.reshape(n_tok, K)
    b1r = b1.reshape(NJ, 1, BH)
    w2r = W2.reshape(NUM_EXPERTS, NJ, BH).transpose(1, 0, 2)
    b2r = b2.reshape(NUM_EXPERTS, 1)

    n_i = n_tok // BT

    out_shapes = (
        jax.ShapeDtypeStruct((NUM_EXPERTS, n_tok), jnp.float32),
        jax.ShapeDtypeStruct((TOP_K, n_tok), jnp.int32),
        jax.ShapeDtypeStruct((TOP_K, n_tok), jnp.float32),
    )

    rw, idx, tkw = pl.pallas_call(
        functools.partial(_router_kernel, n_blocks=n_i),
        grid=(n_i,),
        in_specs=[
            pl.BlockSpec(memory_space=pl.ANY),
            pl.BlockSpec(memory_space=pl.ANY),
            pl.BlockSpec((NJ, 1, BH), lambda i: (0, 0, 0)),
            pl.BlockSpec((NJ, NUM_EXPERTS, BH), lambda i: (0, 0, 0)),
            pl.BlockSpec((NUM_EXPERTS, 1), lambda i: (0, 0)),
        ],
        out_specs=[
            pl.BlockSpec((NUM_EXPERTS, BT), lambda i: (0, i)),
            pl.BlockSpec((TOP_K, BT), lambda i: (0, i)),
            pl.BlockSpec((TOP_K, BT), lambda i: (0, i)),
        ],
        out_shape=out_shapes,
        scratch_shapes=[pltpu.VMEM((BT, HIDDEN), jnp.float32),
                        pltpu.VMEM((NW, BH, HIDDEN), jnp.float32),
                        pltpu.VMEM((NUM_EXPERTS, BT), jnp.float32),
                        pltpu.SemaphoreType.DMA((NQ,)),
                        pltpu.SemaphoreType.DMA((NW, NS))],
        compiler_params=pltpu.CompilerParams(
            dimension_semantics=("arbitrary",)),
        interpret=_INTERPRET,
    )(x2, W1, b1r, w2r, b2r)

    return (rw.T.reshape(B, T, NUM_EXPERTS),
            idx.T.reshape(B, T, TOP_K),
            tkw.T.reshape(B, T, TOP_K))


# speedup vs baseline: 1.1414x; 1.1351x over previous
import functools

import jax
import jax.numpy as jnp
from jax.experimental import pallas as pl
from jax.experimental.pallas import tpu as pltpu

HIDDEN = 4096
NUM_EXPERTS = 64
TOP_K = 8

BT = 2048
BH = 512
NQ = 4
QT = BT // NQ

_INTERPRET = False


def _router_kernel(x_hbm_ref, w1_ref, b1_ref, w2_ref, b2_ref,
                   rw_ref, idx_ref, tkw_ref, x_vmem, acc_ref, sem,
                   *, n_h_blocks):
    i = pl.program_id(0)
    j = pl.program_id(1)

    def qdot(q):
        hq = jax.lax.dot_general(
            x_vmem[pl.ds(q * QT, QT), :], w1_ref[...],
            dimension_numbers=(((1,), (1,)), ((), ())),
            preferred_element_type=jnp.float32)
        hq = jnp.maximum(hq + b1_ref[0, :], 0.0)
        return jax.lax.dot_general(
            w2_ref[...], hq,
            dimension_numbers=(((1,), (1,)), ((), ())),
            preferred_element_type=jnp.float32)

    @pl.when(j == 0)
    def _():
        for q in range(NQ):
            pltpu.make_async_copy(
                x_hbm_ref.at[pl.ds(i * BT + q * QT, QT), :],
                x_vmem.at[pl.ds(q * QT, QT), :], sem.at[q]).start()
        for q in range(NQ):
            pltpu.make_async_copy(
                x_hbm_ref.at[pl.ds(i * BT + q * QT, QT), :],
                x_vmem.at[pl.ds(q * QT, QT), :], sem.at[q]).wait()
            acc_ref[:, pl.ds(q * QT, QT)] = qdot(q)

    @pl.when(j > 0)
    def _():
        h = jax.lax.dot_general(
            x_vmem[...], w1_ref[...],
            dimension_numbers=(((1,), (1,)), ((), ())),
            preferred_element_type=jnp.float32)
        h = jnp.maximum(h + b1_ref[0, :], 0.0)

        part = jax.lax.dot_general(
            w2_ref[...], h,
            dimension_numbers=(((1,), (1,)), ((), ())),
            preferred_element_type=jnp.float32)
        acc_ref[...] += part

    @pl.when(j == n_h_blocks - 1)
    def _():
        logits = acc_ref[...] + b2_ref[:, 0:1]
        m = jnp.max(logits, axis=0, keepdims=True)
        e = jnp.exp(logits - m)
        w = e * (1.0 / jnp.sum(e, axis=0, keepdims=True))
        rw_ref[...] = w

        expert = jax.lax.broadcasted_iota(jnp.int32, (NUM_EXPERTS, BT), 0)
        vals = w
        idx_rows = []
        val_rows = []
        for _ in range(TOP_K):
            mx = jnp.max(vals, axis=0, keepdims=True)
            amx = jnp.min(jnp.where(vals == mx, expert, NUM_EXPERTS),
                          axis=0, keepdims=True)
            idx_rows.append(amx)
            val_rows.append(mx)
            vals = jnp.where(expert == amx, -jnp.inf, vals)
        idx8 = jnp.concatenate(idx_rows, axis=0)
        w8 = jnp.concatenate(val_rows, axis=0)
        idx_ref[...] = idx8
        tkw_ref[...] = w8 * (1.0 / jnp.sum(w8, axis=0, keepdims=True))


def kernel(x, W1, b1, W2, b2):
    B, T, K = x.shape
    n_tok = B * T
    x2 = x.reshape(n_tok, K)
    b1r = b1.reshape(1, K)
    b2r = b2.reshape(NUM_EXPERTS, 1)

    n_i = n_tok // BT
    n_j = K // BH

    out_shapes = (
        jax.ShapeDtypeStruct((NUM_EXPERTS, n_tok), jnp.float32),
        jax.ShapeDtypeStruct((TOP_K, n_tok), jnp.int32),
        jax.ShapeDtypeStruct((TOP_K, n_tok), jnp.float32),
    )

    rw, idx, tkw = pl.pallas_call(
        functools.partial(_router_kernel, n_h_blocks=n_j),
        grid=(n_i, n_j),
        in_specs=[
            pl.BlockSpec(memory_space=pl.ANY),
            pl.BlockSpec((BH, K), lambda i, j: (j, 0)),
            pl.BlockSpec((1, BH), lambda i, j: (0, j)),
            pl.BlockSpec((NUM_EXPERTS, BH), lambda i, j: (0, j)),
            pl.BlockSpec((NUM_EXPERTS, 1), lambda i, j: (0, 0)),
        ],
        out_specs=[
            pl.BlockSpec((NUM_EXPERTS, BT), lambda i, j: (0, i)),
            pl.BlockSpec((TOP_K, BT), lambda i, j: (0, i)),
            pl.BlockSpec((TOP_K, BT), lambda i, j: (0, i)),
        ],
        out_shape=out_shapes,
        scratch_shapes=[pltpu.VMEM((BT, HIDDEN), jnp.float32),
                        pltpu.VMEM((NUM_EXPERTS, BT), jnp.float32),
                        pltpu.SemaphoreType.DMA((NQ,))],
        compiler_params=pltpu.CompilerParams(
            dimension_semantics=("parallel", "arbitrary")),
        interpret=_INTERPRET,
    )(x2, W1, b1r, W2, b2r)

    return (rw.T.reshape(B, T, NUM_EXPERTS),
            idx.T.reshape(B, T, TOP_K),
            tkw.T.reshape(B, T, TOP_K))
